# baseline (speedup 1.0000x reference)
import jax
import jax.numpy as jnp
from jax.experimental import pallas as pl
from jax.experimental.pallas import tpu as pltpu


def _fused_kernel(x_ref, w_ref, a_ref, o_ref, t_ref, acc_ref):
    i = pl.program_id(0)
    k = pl.program_id(1)
    nk = pl.num_programs(1)
    bk = x_ref.shape[0]
    rows = pl.ds(k * bk, bk)

    @pl.when(i == 0)
    def _compute_t_chunk():
        t = jax.lax.dot_general(
            x_ref[...].astype(jnp.bfloat16), w_ref[...].astype(jnp.bfloat16),
            (((1,), (0,)), ((), ())),
            preferred_element_type=jnp.float32)
        t_ref[rows, :] = t.astype(jnp.bfloat16)

    part = jax.lax.dot_general(
        a_ref[...].astype(jnp.bfloat16), t_ref[rows, :],
        (((1,), (0,)), ((), ())),
        preferred_element_type=jnp.float32)

    @pl.when(k == 0)
    def _init():
        acc_ref[...] = part

    @pl.when(k != 0)
    def _accum():
        acc_ref[...] += part

    @pl.when(k == nk - 1)
    def _store():
        o_ref[...] = jnp.maximum(acc_ref[...], 0.0)


def kernel(x_source, neighborhood, W1, W2, W3):
    n, d_in = x_source.shape
    d_out = W1.shape[1]
    bi = min(1024, n)
    bk = min(1024, n)

    def x_index(i, k):
        return (jnp.where(i == 0, k, n // bk - 1), 0)

    out = pl.pallas_call(
        _fused_kernel,
        grid=(n // bi, n // bk),
        in_specs=[pl.BlockSpec((bk, d_in), x_index),
                  pl.BlockSpec((d_in, d_out), lambda i, k: (0, 0)),
                  pl.BlockSpec((bi, bk), lambda i, k: (i, k))],
        out_specs=pl.BlockSpec((bi, d_out), lambda i, k: (i, 0)),
        out_shape=jax.ShapeDtypeStruct((n, d_out), jnp.float32),
        scratch_shapes=[pltpu.VMEM((n, d_out), jnp.bfloat16),
                        pltpu.VMEM((bi, d_out), jnp.float32)],
    )(x_source, W1, neighborhood)
    return out


# speedup vs baseline: 1.2134x; 1.2134x over previous
import jax
import jax.numpy as jnp
from jax.experimental import pallas as pl


def _proj_kernel(x_ref, w_ref, t_ref):
    t = jax.lax.dot_general(
        x_ref[...].astype(jnp.bfloat16), w_ref[...].astype(jnp.bfloat16),
        (((1,), (0,)), ((), ())),
        preferred_element_type=jnp.float32)
    t_ref[...] = t.astype(jnp.bfloat16)


def _spmm_relu_kernel(al_ref, ar_ref, t_ref, o_ref):
    h = al_ref.shape[1]
    accl = jax.lax.dot_general(
        al_ref[...].astype(jnp.bfloat16), t_ref[:h, :],
        (((1,), (0,)), ((), ())),
        preferred_element_type=jnp.float32)
    accr = jax.lax.dot_general(
        ar_ref[...].astype(jnp.bfloat16), t_ref[h:, :],
        (((1,), (0,)), ((), ())),
        preferred_element_type=jnp.float32)
    o_ref[...] = jnp.maximum(accl + accr, 0.0)


def kernel(x_source, neighborhood, W1, W2, W3):
    n, d_in = x_source.shape
    d_out = W1.shape[1]
    bt = min(1024, n)
    bi = min(512, n)

    t = pl.pallas_call(
        _proj_kernel,
        grid=(n // bt,),
        in_specs=[pl.BlockSpec((bt, d_in), lambda i: (i, 0)),
                  pl.BlockSpec((d_in, d_out), lambda i: (0, 0))],
        out_specs=pl.BlockSpec((bt, d_out), lambda i: (i, 0)),
        out_shape=jax.ShapeDtypeStruct((n, d_out), jnp.bfloat16),
    )(x_source, W1)

    out = pl.pallas_call(
        _spmm_relu_kernel,
        grid=(n // bi,),
        in_specs=[pl.BlockSpec((bi, n // 2), lambda i: (i, 0)),
                  pl.BlockSpec((bi, n // 2), lambda i: (i, 1)),
                  pl.BlockSpec((n, d_out), lambda i: (0, 0))],
        out_specs=pl.BlockSpec((bi, d_out), lambda i: (i, 0)),
        out_shape=jax.ShapeDtypeStruct((n, d_out), jnp.float32),
    )(neighborhood, neighborhood, t)
    return out


# speedup vs baseline: 1.2285x; 1.0124x over previous
import jax
import jax.numpy as jnp
from jax.experimental import pallas as pl


def _proj_kernel(x_ref, w_ref, t_ref):
    t = jax.lax.dot_general(
        x_ref[...].astype(jnp.bfloat16), w_ref[...].astype(jnp.bfloat16),
        (((1,), (0,)), ((), ())),
        preferred_element_type=jnp.float32)
    t_ref[...] = t.astype(jnp.bfloat16)


def _spmm_relu_kernel(a_ref, t_ref, o_ref):
    n = a_ref.shape[1]
    chunks = 8
    ck = n // chunks
    acc = None
    for c in range(chunks):
        cols = pl.ds(c * ck, ck)
        part = jax.lax.dot_general(
            a_ref[:, cols].astype(jnp.bfloat16), t_ref[cols, :],
            (((1,), (0,)), ((), ())),
            preferred_element_type=jnp.float32)
        acc = part if acc is None else acc + part
    o_ref[...] = jnp.maximum(acc, 0.0)


def kernel(x_source, neighborhood, W1, W2, W3):
    n, d_in = x_source.shape
    d_out = W1.shape[1]
    bt = min(1024, n)
    bi = min(512, n)

    t = pl.pallas_call(
        _proj_kernel,
        grid=(n // bt,),
        in_specs=[pl.BlockSpec((bt, d_in), lambda i: (i, 0)),
                  pl.BlockSpec((d_in, d_out), lambda i: (0, 0))],
        out_specs=pl.BlockSpec((bt, d_out), lambda i: (i, 0)),
        out_shape=jax.ShapeDtypeStruct((n, d_out), jnp.bfloat16),
    )(x_source, W1)

    out = pl.pallas_call(
        _spmm_relu_kernel,
        grid=(n // bi,),
        in_specs=[pl.BlockSpec((bi, n), lambda i: (i, 0)),
                  pl.BlockSpec((n, d_out), lambda i: (0, 0))],
        out_specs=pl.BlockSpec((bi, d_out), lambda i: (i, 0)),
        out_shape=jax.ShapeDtypeStruct((n, d_out), jnp.float32),
    )(neighborhood, t)
    return out
